# baseline (speedup 1.0000x reference)
import functools
import math

import jax
import jax.numpy as jnp
from jax import lax
from jax.experimental import pallas as pl
from jax.experimental.pallas import tpu as pltpu
from jax.experimental.pallas import tpu_sc as plsc

_NC = 2
_NS = 16
_NW = _NC * _NS
_LANES = 16


@functools.lru_cache(maxsize=None)
def _build(B: int, V: int, D: int, chunk: int, nbuf: int, grow: int):
    assert B % (_NW * chunk) == 0
    assert chunk % grow == 0 and grow <= 128
    assert chunk % 2 == 0 and D % _LANES == 0
    b_per_w = B // _NW
    n_chunks = b_per_w // chunk
    assert n_chunks % nbuf == 0 and n_chunks // nbuf >= 2
    n_gathers = chunk // grow
    scale = math.sqrt(D)
    d_vecs = D // _LANES

    mesh = plsc.VectorSubcoreMesh(core_axis_name="c", subcore_axis_name="s")

    scratch = [pltpu.VMEM((b_per_w,), jnp.int32)]
    scratch += [pltpu.VMEM((chunk, D), jnp.float32) for _ in range(nbuf)]
    scratch += [pltpu.VMEM((chunk // 2, 2 * D), jnp.float32) for _ in range(nbuf)]
    scratch += [pltpu.SemaphoreType.DMA for _ in range(2 * nbuf)]

    @functools.partial(
        pl.kernel,
        out_type=jax.ShapeDtypeStruct((B // 2, 2 * D), jnp.float32),
        mesh=mesh,
        compiler_params=pltpu.CompilerParams(use_tc_tiling_on_sc=False),
        scratch_types=scratch,
    )
    def emb_kernel(x_hbm, table_hbm, out_hbm, idx_v, *bufs_sems):
        gbufs = bufs_sems[:nbuf]
        wbufs = bufs_sems[nbuf : 2 * nbuf]
        gsems = bufs_sems[2 * nbuf : 3 * nbuf]
        wsems = bufs_sems[3 * nbuf :]
        wid = lax.axis_index("s") * _NC + lax.axis_index("c")
        base = wid * b_per_w

        pltpu.sync_copy(x_hbm.at[pl.ds(base, b_per_w)], idx_v)

        def start_gather(j, b):
            for g in range(n_gathers):
                pltpu.async_copy(
                    table_hbm.at[idx_v.at[pl.ds(j * chunk + g * grow, grow)]],
                    gbufs[b].at[pl.ds(g * grow, grow)],
                    gsems[b],
                )

        def wait_gather(j, b):
            for g in range(n_gathers):
                pltpu.make_async_copy(
                    table_hbm.at[idx_v.at[pl.ds(j * chunk + g * grow, grow)]],
                    gbufs[b].at[pl.ds(g * grow, grow)],
                    gsems[b],
                ).wait()

        def scale_pack(b):
            src, dst = gbufs[b], wbufs[b]

            def pack_pair(rp, c):
                for half in range(2):
                    for t in range(d_vecs):
                        dst[rp, pl.ds(half * D + t * _LANES, _LANES)] = (
                            src[2 * rp + half, pl.ds(t * _LANES, _LANES)] * scale
                        )
                return c

            lax.fori_loop(0, chunk // 2, pack_pair, 0, unroll=4)

        def write(j, b):
            pltpu.async_copy(
                wbufs[b],
                out_hbm.at[pl.ds((base + j * chunk) // 2, chunk // 2)],
                wsems[b],
            )

        def wait_write(j, b):
            pltpu.make_async_copy(
                wbufs[b],
                out_hbm.at[pl.ds((base + j * chunk) // 2, chunk // 2)],
                wsems[b],
            ).wait()

        for b in range(nbuf):
            start_gather(b, b)

        def do_group(jj, carry):
            for b in range(nbuf):
                j = jj * nbuf + b
                wait_gather(j, b)
                scale_pack(b)
                write(j, b)
            for b in range(nbuf):
                j = jj * nbuf + b
                wait_write(j, b)
                start_gather(j + nbuf, b)
            return carry

        lax.fori_loop(0, n_chunks // nbuf - 1, do_group, 0)

        for b in range(nbuf):
            j = n_chunks - nbuf + b
            wait_gather(j, b)
            scale_pack(b)
            write(j, b)
        for b in range(nbuf):
            wait_write(n_chunks - nbuf + b, b)

    return emb_kernel


def kernel(x, table):
    V, D = table.shape
    B = x.size
    xf = x.reshape(-1).astype(jnp.int32)
    out = _build(B, V, D, 128, 4, 64)(xf, table)
    return out.reshape(*x.shape, D)
